# Initial kernel scaffold; baseline (speedup 1.0000x reference)
import jax
import jax.numpy as jnp
from jax.experimental import pallas as pl


def kernel(h, edge_index, edge_w, W1, b1, W2):
    raise NotImplementedError("write your pallas kernel here")


# speedup vs baseline: 2.2939x; 2.2939x over previous
import functools

import jax
import jax.numpy as jnp
from jax import lax
from jax.experimental import pallas as pl
from jax.experimental.pallas import tpu as pltpu
from jax.experimental.pallas import tpu_sc as plsc

N_NODES = 10000
N_EDGES = 320000
IN_FEAT = 128
INTER = 256
OUT_FEAT = 128

NC = 2
NS = 16
L = 16
FH = INTER // NC
CHUNK = 128
N_CHUNK = N_EDGES // CHUNK
N_PAD = 10240
ROWS_PER_TILE = N_PAD // NS
NVREG = FH // L

ROW_BLOCK = 1000


def _pre_body(h_ref, w_ref, b_ref, a0_ref, a1_ref):
    hb = h_ref[...]
    a0_ref[...] = jnp.dot(hb, w_ref[0], preferred_element_type=jnp.float32) + b_ref[0]
    a1_ref[...] = jnp.dot(hb, w_ref[1], preferred_element_type=jnp.float32) + b_ref[1]


def _pre(h, W1T_st, b_st):
    return pl.pallas_call(
        _pre_body,
        grid=(N_NODES // ROW_BLOCK,),
        in_specs=[
            pl.BlockSpec((ROW_BLOCK, IN_FEAT), lambda i: (i, 0)),
            pl.BlockSpec((NC, IN_FEAT, FH), lambda i: (0, 0, 0)),
            pl.BlockSpec((NC, 1, FH), lambda i: (0, 0, 0)),
        ],
        out_specs=[pl.BlockSpec((ROW_BLOCK, FH), lambda i: (i, 0))] * 2,
        out_shape=[jax.ShapeDtypeStruct((N_NODES, FH), jnp.float32)] * 2,
    )(h, W1T_st, b_st)


def _post_body(h_ref, n0_ref, n1_ref, wa_ref, w0_ref, w1_ref, o_ref):
    acc = jnp.dot(h_ref[...], wa_ref[...], preferred_element_type=jnp.float32)
    acc = acc + jnp.dot(n0_ref[0], w0_ref[...], preferred_element_type=jnp.float32)
    acc = acc + jnp.dot(n1_ref[0], w1_ref[...], preferred_element_type=jnp.float32)
    o_ref[...] = jnp.maximum(acc, 0.0)


def _post(h, hN, W2aT, W2b0T, W2b1T):
    return pl.pallas_call(
        _post_body,
        grid=(N_NODES // ROW_BLOCK,),
        in_specs=[
            pl.BlockSpec((ROW_BLOCK, IN_FEAT), lambda i: (i, 0)),
            pl.BlockSpec((1, ROW_BLOCK, FH), lambda i: (0, i, 0)),
            pl.BlockSpec((1, ROW_BLOCK, FH), lambda i: (1, i, 0)),
            pl.BlockSpec((IN_FEAT, OUT_FEAT), lambda i: (0, 0)),
            pl.BlockSpec((FH, OUT_FEAT), lambda i: (0, 0)),
            pl.BlockSpec((FH, OUT_FEAT), lambda i: (0, 0)),
        ],
        out_specs=pl.BlockSpec((ROW_BLOCK, OUT_FEAT), lambda i: (i, 0)),
        out_shape=jax.ShapeDtypeStruct((N_NODES, OUT_FEAT), jnp.float32),
    )(h, hN, hN, W2aT, W2b0T, W2b1T)


_sc_mesh = plsc.VectorSubcoreMesh(core_axis_name="c", subcore_axis_name="s")


@functools.partial(
    pl.kernel,
    out_type=jax.ShapeDtypeStruct((NC, N_PAD, FH), jnp.float32),
    mesh=_sc_mesh,
    scratch_types=[
        pltpu.VMEM_SHARED((N_PAD, FH), jnp.float32),
        pltpu.VMEM((CHUNK,), jnp.int32),
        pltpu.VMEM((CHUNK,), jnp.int32),
        pltpu.VMEM((3 * CHUNK,), jnp.float32),
        pltpu.VMEM((3, FH), jnp.float32),
        pltpu.VMEM((CHUNK, FH), jnp.float32),
        pltpu.VMEM((CHUNK, FH), jnp.float32),
        pltpu.SemaphoreType.DMA,
    ],
)
def _sc_segment(a0_hbm, a1_hbm, src_hbm, dst_hbm, ew0_hbm, ew1_hbm, ew2_hbm,
                u_hbm, out_hbm, acc, src_v, dst_v, w_v, u_v, g_v, t_v, sem):
    c = lax.axis_index("c")
    s = lax.axis_index("s")
    zf = jnp.zeros((L,), jnp.float32)

    def _zrow(j, carry):
        for v in range(NVREG):
            t_v[j, pl.ds(v * L, L)] = zf
        return carry
    lax.fori_loop(0, CHUNK, _zrow, 0)
    row0 = s * ROWS_PER_TILE
    for k in range(ROWS_PER_TILE // CHUNK):
        pltpu.sync_copy(t_v, acc.at[pl.ds(row0 + k * CHUNK, CHUNK)])

    pltpu.sync_copy(u_hbm.at[c], u_v)
    u = [[u_v[i, pl.ds(v * L, L)] for v in range(NVREG)] for i in range(3)]

    plsc.subcore_barrier()

    n_my = jnp.where(s < N_CHUNK % NS, N_CHUNK // NS + 1, N_CHUNK // NS)

    def _chunk(ci, carry):
        base = (s + ci * NS) * CHUNK
        pltpu.sync_copy(src_hbm.at[pl.ds(base, CHUNK)], src_v)
        pltpu.sync_copy(dst_hbm.at[pl.ds(base, CHUNK)], dst_v)
        for i, ew_hbm in enumerate((ew0_hbm, ew1_hbm, ew2_hbm)):
            pltpu.sync_copy(ew_hbm.at[pl.ds(base, CHUNK)],
                            w_v.at[pl.ds(i * CHUNK, CHUNK)])

        @pl.when(c == 0)
        def _():
            pltpu.async_copy(a0_hbm.at[src_v], g_v, sem).wait()

        @pl.when(c == 1)
        def _():
            pltpu.async_copy(a1_hbm.at[src_v], g_v, sem).wait()

        def _group(g, gcarry):
            w0g = w_v[pl.ds(g * L, L)]
            w1g = w_v[pl.ds(CHUNK + g * L, L)]
            w2g = w_v[pl.ds(2 * CHUNK + g * L, L)]
            for jj in range(L):
                j = g * L + jj
                w0 = jnp.full((L,), w0g[jj], jnp.float32)
                w1 = jnp.full((L,), w1g[jj], jnp.float32)
                w2 = jnp.full((L,), w2g[jj], jnp.float32)
                for v in range(NVREG):
                    gg = g_v[j, pl.ds(v * L, L)]
                    t = gg + w0 * u[0][v] + w1 * u[1][v] + w2 * u[2][v]
                    t_v[j, pl.ds(v * L, L)] = jnp.maximum(t, 0.0)
            return gcarry
        lax.fori_loop(0, CHUNK // L, _group, 0)

        pltpu.sync_copy(t_v, acc.at[dst_v], add=True)
        return carry
    lax.fori_loop(0, n_my, _chunk, 0)

    plsc.subcore_barrier()

    pltpu.sync_copy(acc.at[pl.ds(row0, ROWS_PER_TILE)],
                    out_hbm.at[c, pl.ds(row0, ROWS_PER_TILE)])


def kernel(h, edge_index, edge_w, W1, b1, W2):
    src = edge_index[0].astype(jnp.int32)
    dst = edge_index[1].astype(jnp.int32)
    ew0, ew1, ew2 = edge_w[:, 0], edge_w[:, 1], edge_w[:, 2]

    W1h = W1[:, :IN_FEAT]
    W1T_st = jnp.stack([W1h[:FH].T, W1h[FH:].T])
    b_st = jnp.stack([b1[:FH], b1[FH:]])[:, None, :]
    W1w = W1[:, IN_FEAT:]
    U_st = jnp.stack([W1w[:FH].T, W1w[FH:].T])

    W2aT = W2[:, :IN_FEAT].T
    W2b0T = W2[:, IN_FEAT:IN_FEAT + FH].T
    W2b1T = W2[:, IN_FEAT + FH:].T

    A0, A1 = _pre(h, W1T_st, b_st)
    hN = _sc_segment(A0, A1, src, dst, ew0, ew1, ew2, U_st)
    return _post(h, hN, W2aT, W2b0T, W2b1T)


# speedup vs baseline: 3.2148x; 1.4015x over previous
import functools

import jax
import jax.numpy as jnp
from jax import lax
from jax.experimental import pallas as pl
from jax.experimental.pallas import tpu as pltpu
from jax.experimental.pallas import tpu_sc as plsc

N_NODES = 10000
N_EDGES = 320000
IN_FEAT = 128
INTER = 256
OUT_FEAT = 128

NC = 2
NS = 16
L = 16
FH = INTER // NC
CHUNK = 128
N_CHUNK = N_EDGES // CHUNK
N_PAD = 10240
ROWS_PER_TILE = N_PAD // NS
NVREG = FH // L

ROW_BLOCK = 1000


def _pre_body(h_ref, w_ref, b_ref, a0_ref, a1_ref):
    hb = h_ref[...]
    a0_ref[...] = jnp.dot(hb, w_ref[0], preferred_element_type=jnp.float32) + b_ref[0]
    a1_ref[...] = jnp.dot(hb, w_ref[1], preferred_element_type=jnp.float32) + b_ref[1]


def _pre(h, W1T_st, b_st):
    return pl.pallas_call(
        _pre_body,
        grid=(N_NODES // ROW_BLOCK,),
        in_specs=[
            pl.BlockSpec((ROW_BLOCK, IN_FEAT), lambda i: (i, 0)),
            pl.BlockSpec((NC, IN_FEAT, FH), lambda i: (0, 0, 0)),
            pl.BlockSpec((NC, 1, FH), lambda i: (0, 0, 0)),
        ],
        out_specs=[pl.BlockSpec((ROW_BLOCK, FH), lambda i: (i, 0))] * 2,
        out_shape=[jax.ShapeDtypeStruct((N_NODES, FH), jnp.float32)] * 2,
    )(h, W1T_st, b_st)


def _post_body(h_ref, n0_ref, n1_ref, wa_ref, w0_ref, w1_ref, o_ref):
    acc = jnp.dot(h_ref[...], wa_ref[...], preferred_element_type=jnp.float32)
    acc = acc + jnp.dot(n0_ref[0], w0_ref[...], preferred_element_type=jnp.float32)
    acc = acc + jnp.dot(n1_ref[0], w1_ref[...], preferred_element_type=jnp.float32)
    o_ref[...] = jnp.maximum(acc, 0.0)


def _post(h, hN, W2aT, W2b0T, W2b1T):
    return pl.pallas_call(
        _post_body,
        grid=(N_NODES // ROW_BLOCK,),
        in_specs=[
            pl.BlockSpec((ROW_BLOCK, IN_FEAT), lambda i: (i, 0)),
            pl.BlockSpec((1, ROW_BLOCK, FH), lambda i: (0, i, 0)),
            pl.BlockSpec((1, ROW_BLOCK, FH), lambda i: (1, i, 0)),
            pl.BlockSpec((IN_FEAT, OUT_FEAT), lambda i: (0, 0)),
            pl.BlockSpec((FH, OUT_FEAT), lambda i: (0, 0)),
            pl.BlockSpec((FH, OUT_FEAT), lambda i: (0, 0)),
        ],
        out_specs=pl.BlockSpec((ROW_BLOCK, OUT_FEAT), lambda i: (i, 0)),
        out_shape=jax.ShapeDtypeStruct((N_NODES, OUT_FEAT), jnp.float32),
    )(h, hN, hN, W2aT, W2b0T, W2b1T)


_sc_mesh = plsc.VectorSubcoreMesh(core_axis_name="c", subcore_axis_name="s")


@functools.partial(
    pl.kernel,
    out_type=jax.ShapeDtypeStruct((NC, N_PAD, FH), jnp.float32),
    mesh=_sc_mesh,
    scratch_types=[
        pltpu.VMEM_SHARED((N_PAD, FH), jnp.float32),
        pltpu.VMEM((2, CHUNK), jnp.int32),
        pltpu.VMEM((2, CHUNK), jnp.int32),
        pltpu.VMEM((2, 4 * CHUNK), jnp.float32),
        pltpu.VMEM((3, FH), jnp.float32),
        pltpu.VMEM((2, CHUNK, FH), jnp.float32),
        pltpu.SemaphoreType.DMA,
        pltpu.SemaphoreType.DMA,
        pltpu.SemaphoreType.DMA,
        pltpu.SemaphoreType.DMA,
        pltpu.SemaphoreType.DMA,
        pltpu.SemaphoreType.DMA,
        pltpu.SemaphoreType.DMA,
        pltpu.SemaphoreType.DMA,
    ],
)
def _sc_segment(a0_hbm, a1_hbm, src_hbm, dst_hbm, ew_hbm,
                u_hbm, out_hbm, acc, src_v, dst_v, w_v, u_v, g_v,
                si0, si1, sg0, sg1, ss0, ss1, sd0, sd1):
    c = lax.axis_index("c")
    s = lax.axis_index("s")
    zf = jnp.zeros((L,), jnp.float32)
    si = (si0, si1)
    sg = (sg0, sg1)
    ss = (ss0, ss1)
    sd = (sd0, sd1)

    def _zrow(j, carry):
        for v in range(NVREG):
            g_v[0, j, pl.ds(v * L, L)] = zf
        return carry
    lax.fori_loop(0, CHUNK, _zrow, 0)
    row0 = s * ROWS_PER_TILE
    for k in range(ROWS_PER_TILE // CHUNK):
        pltpu.sync_copy(g_v.at[0], acc.at[pl.ds(row0 + k * CHUNK, CHUNK)])

    pltpu.sync_copy(u_hbm.at[c], u_v)
    u = [[u_v[i, pl.ds(v * L, L)] for v in range(NVREG)] for i in range(3)]

    plsc.subcore_barrier()

    n_my = jnp.where(s < N_CHUNK % NS, N_CHUNK // NS + 1, N_CHUNK // NS)

    def chunk_base(j):
        return (s + j * NS) * CHUNK

    def fire_idx(j, b):
        base = chunk_base(j)
        pltpu.async_copy(src_hbm.at[pl.ds(base, CHUNK)], src_v.at[b], si[b])
        pltpu.async_copy(ew_hbm.at[pl.ds(4 * base, 4 * CHUNK)], w_v.at[b], si[b])

    def wait_idx(b):
        pltpu.make_async_copy(src_hbm.at[pl.ds(0, CHUNK)], src_v.at[b], si[b]).wait()
        pltpu.make_async_copy(ew_hbm.at[pl.ds(0, 4 * CHUNK)], w_v.at[b], si[b]).wait()

    def fire_gather(b):
        @pl.when(c == 0)
        def _():
            pltpu.async_copy(a0_hbm.at[src_v.at[b]], g_v.at[b], sg[b])

        @pl.when(c == 1)
        def _():
            pltpu.async_copy(a1_hbm.at[src_v.at[b]], g_v.at[b], sg[b])

    def wait_gather(b):
        pltpu.make_async_copy(a0_hbm.at[src_v.at[b]], g_v.at[b], sg[b]).wait()

    def fire_dst(j, b):
        pltpu.async_copy(dst_hbm.at[pl.ds(chunk_base(j), CHUNK)], dst_v.at[b], sd[b])

    def wait_dst(b):
        pltpu.make_async_copy(dst_hbm.at[pl.ds(0, CHUNK)], dst_v.at[b], sd[b]).wait()

    def fire_scatter(b):
        pltpu.async_copy(g_v.at[b], acc.at[dst_v.at[b]], ss[b], add=True)

    def wait_scatter(b):
        pltpu.make_async_copy(g_v.at[b], acc.at[dst_v.at[b]], ss[b]).wait()

    def compute(b):
        def _group(g, gcarry):
            j0 = g * L
            wq = [w_v[b, pl.ds(4 * j0 + 16 * k, L)] for k in range(4)]
            for jj in range(L):
                q, r = divmod(jj, 4)
                w0 = jnp.full((L,), wq[q][4 * r + 0], jnp.float32)
                w1 = jnp.full((L,), wq[q][4 * r + 1], jnp.float32)
                w2 = jnp.full((L,), wq[q][4 * r + 2], jnp.float32)
                j = j0 + jj
                for v in range(NVREG):
                    gg = g_v[b, j, pl.ds(v * L, L)]
                    t = gg + w0 * u[0][v] + w1 * u[1][v] + w2 * u[2][v]
                    g_v[b, j, pl.ds(v * L, L)] = jnp.maximum(t, 0.0)
            return gcarry
        lax.fori_loop(0, CHUNK // L, _group, 0)

    def step(j, b):
        wait_gather(b)

        @pl.when(j + 1 < n_my)
        def _():
            wait_idx(1 - b)

            @pl.when(j >= 1)
            def _():
                wait_scatter(1 - b)
            fire_gather(1 - b)

        fire_dst(j, b)
        compute(b)
        wait_dst(b)
        fire_scatter(b)

        @pl.when(j + 2 < n_my)
        def _():
            fire_idx(j + 2, b)

    fire_idx(0, 0)
    fire_idx(1, 1)
    wait_idx(0)
    fire_gather(0)

    def _pair(p, carry):
        j0 = 2 * p
        step(j0, 0)

        @pl.when(j0 + 1 < n_my)
        def _():
            step(j0 + 1, 1)
        return carry
    lax.fori_loop(0, (n_my + 1) // 2, _pair, 0)

    wait_scatter(0)
    wait_scatter(1)

    plsc.subcore_barrier()

    pltpu.sync_copy(acc.at[pl.ds(row0, ROWS_PER_TILE)],
                    out_hbm.at[c, pl.ds(row0, ROWS_PER_TILE)])


def kernel(h, edge_index, edge_w, W1, b1, W2):
    src = edge_index[0].astype(jnp.int32)
    dst = edge_index[1].astype(jnp.int32)
    ew4 = jnp.pad(edge_w, ((0, 0), (0, 1))).reshape(-1)

    W1h = W1[:, :IN_FEAT]
    W1T_st = jnp.stack([W1h[:FH].T, W1h[FH:].T])
    b_st = jnp.stack([b1[:FH], b1[FH:]])[:, None, :]
    W1w = W1[:, IN_FEAT:]
    U_st = jnp.stack([W1w[:FH].T, W1w[FH:].T])

    W2aT = W2[:, :IN_FEAT].T
    W2b0T = W2[:, IN_FEAT:IN_FEAT + FH].T
    W2b1T = W2[:, IN_FEAT + FH:].T

    A0, A1 = _pre(h, W1T_st, b_st)
    hN = _sc_segment(A0, A1, src, dst, ew4, U_st)
    return _post(h, hN, W2aT, W2b0T, W2b1T)
